# Initial kernel scaffold; baseline (speedup 1.0000x reference)
import jax
import jax.numpy as jnp
from jax.experimental import pallas as pl


def kernel(z, nodes_id, records):
    raise NotImplementedError("write your pallas kernel here")


# speedup vs baseline: 1.6119x; 1.6119x over previous
import jax
import jax.numpy as jnp
from jax import lax
from jax.experimental import pallas as pl
from jax.experimental.pallas import tpu as pltpu
from jax.experimental.pallas import tpu_sc as plsc

_BETA = 0.2

_N = 100000
_K = 50000
_D = 128
_NTILES = 32
_CNT = 3200
_ROWS = 160
_LANES = 16


def _body(z_hbm, nid_hbm, rec_hbm, out_hbm, idx_v, w_v, wi_v, zbuf, rbuf, sem):
    c = lax.axis_index("c")
    s = lax.axis_index("s")
    wid = c * 16 + s
    lo = wid * _CNT
    cnt = jnp.minimum(_CNT, _N - lo)

    pltpu.sync_copy(nid_hbm, idx_v)

    lanes = lax.iota(jnp.int32, _LANES)

    def memset(k, carry):
        w_v[pl.ds(k * _LANES, _LANES)] = jnp.full((_LANES,), -1, jnp.int32)
        return carry
    lax.fori_loop(0, _CNT // _LANES, memset, 0)

    def scan(v, carry):
        idx = idx_v[pl.ds(v * _LANES, _LANES)]
        rel = idx - lo
        m = (rel >= 0) & (rel < cnt)
        _, lastm = plsc.scan_count(rel, mask=m)
        sm = m & lastm
        relc = jnp.where(sm, rel, 0)
        iv = v * _LANES + lanes
        plsc.store_scatter(w_v, [relc], iv, mask=sm)
        return carry
    lax.fori_loop(0, _K // _LANES, scan, 0)

    def mkwi(k, carry):
        wv = w_v[pl.ds(k * _LANES, _LANES)]
        dummy = (k * _LANES + lanes) & 8191
        wi_v[pl.ds(k * _LANES, _LANES)] = jnp.where(wv >= 0, wv, dummy)
        return carry
    lax.fori_loop(0, _CNT // _LANES, mkwi, 0)

    def blk(b, carry):
        base = lo + b * _ROWS
        pltpu.sync_copy(z_hbm.at[pl.ds(base, _ROWS)], zbuf)
        pltpu.async_copy(
            rec_hbm.at[wi_v.at[pl.ds(b * _ROWS, _ROWS)]], rbuf, sem
        ).wait()

        def grp(g, gcarry):
            wv = w_v[pl.ds(b * _ROWS + g * _LANES, _LANES)]
            fv = jnp.where(
                wv >= 0, jnp.float32(1.0 - _BETA), jnp.float32(0.0)
            )
            for j in range(_LANES):
                r = g * _LANES + j
                f = fv[j]
                for q in range(_D // _LANES):
                    zv = zbuf[r, pl.ds(q * _LANES, _LANES)]
                    rv = rbuf[r, pl.ds(q * _LANES, _LANES)]
                    zbuf[r, pl.ds(q * _LANES, _LANES)] = zv + f * (rv - zv)
            return gcarry
        lax.fori_loop(0, _ROWS // _LANES, grp, 0)

        pltpu.sync_copy(zbuf, out_hbm.at[pl.ds(base, _ROWS)])
        return carry
    lax.fori_loop(0, cnt // _ROWS, blk, 0)


def kernel(z, nodes_id, records):
    mesh = plsc.VectorSubcoreMesh(
        core_axis_name="c", subcore_axis_name="s", num_cores=2, num_subcores=16
    )
    return pl.kernel(
        _body,
        out_type=jax.ShapeDtypeStruct((_N, _D), jnp.float32),
        mesh=mesh,
        compiler_params=pltpu.CompilerParams(needs_layout_passes=False),
        scratch_types=[
            pltpu.VMEM((_K,), jnp.int32),
            pltpu.VMEM((_CNT,), jnp.int32),
            pltpu.VMEM((_CNT,), jnp.int32),
            pltpu.VMEM((_ROWS, _D), jnp.float32),
            pltpu.VMEM((_ROWS, _D), jnp.float32),
            pltpu.SemaphoreType.DMA,
        ],
    )(z, nodes_id, records)
